# baseline (speedup 1.0000x reference)
import jax
import jax.numpy as jnp
from jax import lax
from jax.experimental import pallas as pl
from jax.experimental.pallas import tpu as pltpu
from jax.experimental.pallas import tpu_sc as plsc

MAXLEN = 2048
EMBED_DIM = 128
BATCH = 32

NUM_CORES = 2
NUM_SUBCORES = 16
LANES = 16
NW = NUM_CORES * NUM_SUBCORES
P = MAXLEN // NW
GB = 4
NGRP = BATCH // GB
NBUF = 3 * GB
XCOL = 128


def _emb_body(x_hbm, tok_hbm, pos_hbm, out_hbm,
              idx_all, rows_v, pos_v, gsem, ssem, psem):
    wid = lax.axis_index("s") * NUM_CORES + lax.axis_index("c")
    pbase = wid * P
    cbase = pl.multiple_of((wid // 2) * XCOL, XCOL)
    off = (wid % 2) * P

    pos_cp = pltpu.async_copy(pos_hbm.at[pl.ds(pbase, P)], pos_v, psem)
    pltpu.sync_copy(x_hbm.at[:, pl.ds(cbase, XCOL)], idx_all)

    def gather(b):
        return pltpu.make_async_copy(
            tok_hbm.at[idx_all.at[b, pl.ds(off, P)]],
            rows_v.at[b % NBUF], gsem.at[b % NBUF])

    def store(b):
        return pltpu.make_async_copy(
            rows_v.at[b % NBUF],
            out_hbm.at[pl.ds(b * MAXLEN + pbase, P)],
            ssem.at[b % NBUF])

    for b in range(2 * GB):
        gather(b).start()
    pos_cp.wait()

    for g in range(NGRP):
        b0 = g * GB
        for i in range(GB):
            gather(b0 + i).wait()

        def add_body(r, carry):
            for j in range(EMBED_DIM // LANES):
                sl = pl.ds(j * LANES, LANES)
                pv = pos_v[r, sl]
                for i in range(GB):
                    s = (b0 + i) % NBUF
                    rows_v[s, r, sl] = rows_v[s, r, sl] + pv
            return carry

        lax.fori_loop(0, P, add_body, 0)
        for i in range(GB):
            store(b0 + i).start()
        if g + 2 < NGRP:
            for i in range(GB):
                nb = (g + 2) * GB + i
                if nb >= NBUF:
                    store(nb - NBUF).wait()
                gather(nb).start()

    for b in range(BATCH - NBUF, BATCH):
        store(b).wait()


def kernel(x, token_table, pos_table):
    x32 = x.astype(jnp.int32)
    mesh = plsc.VectorSubcoreMesh(core_axis_name="c", subcore_axis_name="s")
    f = pl.kernel(
        _emb_body,
        mesh=mesh,
        out_type=jax.ShapeDtypeStruct((BATCH * MAXLEN, EMBED_DIM), jnp.float32),
        scratch_types=[
            pltpu.VMEM((BATCH, XCOL), jnp.int32),
            pltpu.VMEM((NBUF, P, EMBED_DIM), jnp.float32),
            pltpu.VMEM((P, EMBED_DIM), jnp.float32),
            pltpu.SemaphoreType.DMA((NBUF,)),
            pltpu.SemaphoreType.DMA((NBUF,)),
            pltpu.SemaphoreType.DMA,
        ],
    )
    out = f(x32, token_table, pos_table)
    return out.reshape(BATCH, MAXLEN, EMBED_DIM)


# speedup vs baseline: 1.0216x; 1.0216x over previous
import jax
import jax.numpy as jnp
from jax import lax
from jax.experimental import pallas as pl
from jax.experimental.pallas import tpu as pltpu
from jax.experimental.pallas import tpu_sc as plsc

MAXLEN = 2048
EMBED_DIM = 128
BATCH = 32

NUM_CORES = 2
NUM_SUBCORES = 16
LANES = 16
NW = NUM_CORES * NUM_SUBCORES
P = MAXLEN // NW
NBUF = 8
LOOKAHEAD = 4
XCOL = 128


def _emb_body(x_hbm, tok_hbm, pos_hbm, out_hbm,
              idx_all, rows_v, pos_v, gsem, ssem, psem):
    wid = lax.axis_index("s") * NUM_CORES + lax.axis_index("c")
    pbase = wid * P
    cbase = pl.multiple_of((wid // 2) * XCOL, XCOL)
    off = (wid % 2) * P

    pos_cp = pltpu.async_copy(pos_hbm.at[pl.ds(pbase, P)], pos_v, psem)
    pltpu.sync_copy(x_hbm.at[:, pl.ds(cbase, XCOL)], idx_all)

    def gather(b):
        return pltpu.make_async_copy(
            tok_hbm.at[idx_all.at[b, pl.ds(off, P)]],
            rows_v.at[b % NBUF], gsem.at[b % NBUF])

    def store(b):
        return pltpu.make_async_copy(
            rows_v.at[b % NBUF],
            out_hbm.at[pl.ds(b * MAXLEN + pbase, P)],
            ssem.at[b % NBUF])

    for b in range(LOOKAHEAD):
        gather(b).start()
    pos_cp.wait()

    for b in range(BATCH):
        s = b % NBUF
        gather(b).wait()

        def add_body(r, carry):
            for j in range(EMBED_DIM // LANES):
                sl = pl.ds(j * LANES, LANES)
                rows_v[s, r, sl] = rows_v[s, r, sl] + pos_v[r, sl]
            return carry

        lax.fori_loop(0, P, add_body, 0)
        store(b).start()
        if b + LOOKAHEAD < BATCH:
            nb = b + LOOKAHEAD
            if nb >= NBUF:
                store(nb - NBUF).wait()
            gather(nb).start()

    for b in range(BATCH - NBUF, BATCH):
        store(b).wait()


def kernel(x, token_table, pos_table):
    x32 = x.astype(jnp.int32)
    mesh = plsc.VectorSubcoreMesh(core_axis_name="c", subcore_axis_name="s")
    f = pl.kernel(
        _emb_body,
        mesh=mesh,
        out_type=jax.ShapeDtypeStruct((BATCH * MAXLEN, EMBED_DIM), jnp.float32),
        scratch_types=[
            pltpu.VMEM((BATCH, XCOL), jnp.int32),
            pltpu.VMEM((NBUF, P, EMBED_DIM), jnp.float32),
            pltpu.VMEM((P, EMBED_DIM), jnp.float32),
            pltpu.SemaphoreType.DMA((NBUF,)),
            pltpu.SemaphoreType.DMA((NBUF,)),
            pltpu.SemaphoreType.DMA,
        ],
    )
    out = f(x32, token_table, pos_table)
    return out.reshape(BATCH, MAXLEN, EMBED_DIM)


# speedup vs baseline: 1.0490x; 1.0268x over previous
import jax
import jax.numpy as jnp
from jax import lax
from jax.experimental import pallas as pl
from jax.experimental.pallas import tpu as pltpu
from jax.experimental.pallas import tpu_sc as plsc

MAXLEN = 2048
EMBED_DIM = 128
BATCH = 32

NUM_CORES = 2
NUM_SUBCORES = 16
LANES = 16
NW = NUM_CORES * NUM_SUBCORES
P = MAXLEN // NW
NBUF = 14
LOOKAHEAD = 8
XCOL = 128


def _emb_body(x_hbm, tok_hbm, pos_hbm, out_hbm,
              idx_all, rows_v, pos_v, gsem, ssem, psem):
    wid = lax.axis_index("s") * NUM_CORES + lax.axis_index("c")
    pbase = wid * P
    cbase = pl.multiple_of((wid // 2) * XCOL, XCOL)
    off = (wid % 2) * P

    pos_cp = pltpu.async_copy(pos_hbm.at[pl.ds(pbase, P)], pos_v, psem)
    pltpu.sync_copy(x_hbm.at[:, pl.ds(cbase, XCOL)], idx_all)

    def gather(b):
        return pltpu.make_async_copy(
            tok_hbm.at[idx_all.at[b, pl.ds(off, P)]],
            rows_v.at[b % NBUF], gsem.at[b % NBUF])

    def store(b):
        return pltpu.make_async_copy(
            rows_v.at[b % NBUF],
            out_hbm.at[pl.ds(b * MAXLEN + pbase, P)],
            ssem.at[b % NBUF])

    for b in range(LOOKAHEAD):
        gather(b).start()
    pos_cp.wait()

    for b in range(BATCH):
        s = b % NBUF
        gather(b).wait()

        def add_body(r, carry):
            for j in range(EMBED_DIM // LANES):
                sl = pl.ds(j * LANES, LANES)
                rows_v[s, r, sl] = rows_v[s, r, sl] + pos_v[r, sl]
            return carry

        lax.fori_loop(0, P, add_body, 0)
        store(b).start()
        if b + LOOKAHEAD < BATCH:
            nb = b + LOOKAHEAD
            if nb >= NBUF:
                store(nb - NBUF).wait()
            gather(nb).start()

    for b in range(BATCH - NBUF, BATCH):
        store(b).wait()


def kernel(x, token_table, pos_table):
    x32 = x.astype(jnp.int32)
    mesh = plsc.VectorSubcoreMesh(core_axis_name="c", subcore_axis_name="s")
    f = pl.kernel(
        _emb_body,
        mesh=mesh,
        out_type=jax.ShapeDtypeStruct((BATCH * MAXLEN, EMBED_DIM), jnp.float32),
        scratch_types=[
            pltpu.VMEM((BATCH, XCOL), jnp.int32),
            pltpu.VMEM((NBUF, P, EMBED_DIM), jnp.float32),
            pltpu.VMEM((P, EMBED_DIM), jnp.float32),
            pltpu.SemaphoreType.DMA((NBUF,)),
            pltpu.SemaphoreType.DMA((NBUF,)),
            pltpu.SemaphoreType.DMA,
        ],
    )
    out = f(x32, token_table, pos_table)
    return out.reshape(BATCH, MAXLEN, EMBED_DIM)
